# baseline (speedup 1.0000x reference)
import functools

import jax
import jax.numpy as jnp
from jax import lax
from jax.experimental import pallas as pl
from jax.experimental.pallas import tpu as pltpu
from jax.experimental.pallas import tpu_sc as plsc

_ROWS = 4 * 1024 * 1024
_CLASSES = 4
_SEG_ROWS = 1024 * 1024
_FLAT = _ROWS * _CLASSES
_NW = 32
_PER_W = _FLAT // _NW
_CHUNK = 32768
_NCHUNK = _PER_W // _CHUNK
_LANES = 16


def _build_one_hot_flat():
    mesh = plsc.VectorSubcoreMesh(core_axis_name="c", subcore_axis_name="s")

    @functools.partial(
        pl.kernel,
        mesh=mesh,
        out_type=jax.ShapeDtypeStruct((_FLAT,), jnp.int32),
        scratch_types=[
            pltpu.VMEM((_CHUNK,), jnp.int32),
            pltpu.SemaphoreType.DMA,
        ],
    )
    def k(out_hbm, buf_v, sem):
        wid = lax.axis_index("s") * 2 + lax.axis_index("c")
        lid = wid // (_NW // _CLASSES)
        ones = jnp.full((_LANES,), 1, dtype=jnp.int32)
        zeros = jnp.full((_LANES,), 0, dtype=jnp.int32)

        def fill(i, _):
            klass = lax.rem(i, 32) // (128 // _LANES)
            buf_v[pl.ds(i * _LANES, _LANES)] = jnp.where(
                klass == lid, ones, zeros)
            return 0

        lax.fori_loop(0, _CHUNK // _LANES, fill, 0, unroll=8)

        base = wid * _PER_W
        copies = [
            pltpu.make_async_copy(
                buf_v, out_hbm.at[pl.ds(base + c * _CHUNK, _CHUNK)], sem)
            for c in range(_NCHUNK)
        ]
        for cp in copies:
            cp.start()
        for cp in copies:
            cp.wait()

    return k()


def kernel(w0, w1, w2, w3, y):
    flat = _build_one_hot_flat()
    one_hot = (flat.reshape(_ROWS // 128, _CLASSES, 128)
               .transpose(0, 2, 1)
               .reshape(_ROWS, _CLASSES)
               .astype(jnp.int64))
    return (one_hot, y)


# speedup vs baseline: 1.0089x; 1.0089x over previous
import functools

import jax
import jax.numpy as jnp
from jax import lax
from jax.experimental import pallas as pl
from jax.experimental.pallas import tpu as pltpu
from jax.experimental.pallas import tpu_sc as plsc

_ROWS = 4 * 1024 * 1024
_CLASSES = 4
_SEG_ROWS = 1024 * 1024
_FLAT = _ROWS * _CLASSES
_NW = 32
_PER_W = _FLAT // _NW
_CHUNK = 16384
_NCHUNK = _PER_W // _CHUNK
_LANES = 16


def _build_one_hot_flat():
    mesh = plsc.VectorSubcoreMesh(core_axis_name="c", subcore_axis_name="s")

    @functools.partial(
        pl.kernel,
        mesh=mesh,
        out_type=jax.ShapeDtypeStruct((_FLAT,), jnp.int32),
        scratch_types=[
            pltpu.VMEM((_CHUNK,), jnp.int32),
            pltpu.SemaphoreType.DMA,
        ],
    )
    def k(out_hbm, buf_v, sem):
        wid = lax.axis_index("s") * 2 + lax.axis_index("c")
        lid = wid // (_NW // _CLASSES)
        ones = jnp.full((_LANES,), 1, dtype=jnp.int32)
        zeros = jnp.full((_LANES,), 0, dtype=jnp.int32)

        def fill(i, _):
            klass = lax.rem(i, 32) // (128 // _LANES)
            buf_v[pl.ds(i * _LANES, _LANES)] = jnp.where(
                klass == lid, ones, zeros)
            return 0

        lax.fori_loop(0, _CHUNK // _LANES, fill, 0, unroll=8)

        base = wid * _PER_W
        copies = [
            pltpu.make_async_copy(
                buf_v, out_hbm.at[pl.ds(base + c * _CHUNK, _CHUNK)], sem)
            for c in range(_NCHUNK)
        ]
        for cp in copies:
            cp.start()
        for cp in copies:
            cp.wait()

    return k()


def kernel(w0, w1, w2, w3, y):
    flat = _build_one_hot_flat()
    one_hot = (flat.reshape(_ROWS // 128, _CLASSES, 128)
               .transpose(0, 2, 1)
               .reshape(_ROWS, _CLASSES)
               .astype(jnp.int64))
    return (one_hot, y)
